# Initial kernel scaffold; baseline (speedup 1.0000x reference)
import jax
import jax.numpy as jnp
from jax.experimental import pallas as pl


def kernel(x, edge_d, params, edge_index):
    raise NotImplementedError("write your pallas kernel here")


# speedup vs baseline: 2.2512x; 2.2512x over previous
import functools

import jax
import jax.numpy as jnp
from jax import lax
from jax.experimental import pallas as pl
from jax.experimental.pallas import tpu as pltpu
from jax.experimental.pallas import tpu_sc as plsc


def _silu(v):
    return v * jax.nn.sigmoid(v)


def _dot(a, b):
    return lax.dot_general(a, b, (((1,), (0,)), ((), ())),
                           preferred_element_type=jnp.float32)


def _node_in_body(x_ref, win_ref, bin_ref, feat_ref):
    feat_ref[...] = _silu(_silu(_dot(x_ref[...], win_ref[...]) + bin_ref[...]))


def _edge_common(gs, gd, d, w1_ref, b1_ref, w2_ref, b2_ref, ws_ref, bs_ref,
                 dnew_ref, gated_ref):
    mi = jnp.concatenate([gs, gd, d], axis=-1)
    m = _silu(_dot(mi, w1_ref[...]) + b1_ref[...])
    m = _silu(_dot(m, w2_ref[...]) + b2_ref[...])
    dnew_ref[...] = d + m
    ew = jax.nn.sigmoid(_dot(m, ws_ref[...]) + bs_ref[...])
    gated_ref[...] = m * ew


def _edge0_body(gs_ref, gd_ref, ed_ref, we_ref, be_ref,
                w1_ref, b1_ref, w2_ref, b2_ref, ws_ref, bs_ref,
                dnew_ref, gated_ref):
    d = _silu(_silu(ed_ref[...] * we_ref[...] + be_ref[...]))
    _edge_common(gs_ref[...], gd_ref[...], d, w1_ref, b1_ref, w2_ref, b2_ref,
                 ws_ref, bs_ref, dnew_ref, gated_ref)


def _edge_body(gs_ref, gd_ref, d_ref, w1_ref, b1_ref, w2_ref, b2_ref,
               ws_ref, bs_ref, dnew_ref, gated_ref):
    _edge_common(gs_ref[...], gd_ref[...], d_ref[...], w1_ref, b1_ref,
                 w2_ref, b2_ref, ws_ref, bs_ref, dnew_ref, gated_ref)


def _update_body(ms_ref, feat_ref, u1_ref, bu1_ref, u2_ref, bu2_ref,
                 feat_out_ref):
    feat = feat_ref[...]
    msum = ms_ref[0] + ms_ref[1]
    h = _silu(_dot(msum + feat, u1_ref[...]) + bu1_ref[...])
    feat_out_ref[...] = _dot(h, u2_ref[...]) + bu2_ref[...] + feat


def _final_body(ms_ref, feat_ref, u1_ref, bu1_ref, u2_ref, bu2_ref,
                o1_ref, bo1_ref, o2_ref, bo2_ref, out_ref):
    feat = feat_ref[...]
    msum = ms_ref[0] + ms_ref[1]
    h = _silu(_dot(msum + feat, u1_ref[...]) + bu1_ref[...])
    fnew = _dot(h, u2_ref[...]) + bu2_ref[...] + feat
    h2 = _silu(_dot(fnew, o1_ref[...]) + bo1_ref[...])
    out_ref[...] = _dot(h2, o2_ref[...]) + bo2_ref[...]


def _wspec(shape):
    nd = len(shape)
    return pl.BlockSpec(shape, lambda i: (0,) * nd)


def _make_gather(E, N, H, n_workers, chunk):
    epw = E // n_workers
    nchunk = epw // chunk
    mesh = plsc.VectorSubcoreMesh(core_axis_name="c", subcore_axis_name="s")

    @functools.partial(
        pl.kernel,
        out_type=[jax.ShapeDtypeStruct((E, H), jnp.float32),
                  jax.ShapeDtypeStruct((E, H), jnp.float32)],
        mesh=mesh,
        scratch_types=[
            pltpu.VMEM((chunk,), jnp.int32),
            pltpu.VMEM((chunk, H), jnp.float32),
            pltpu.VMEM((chunk,), jnp.int32),
            pltpu.VMEM((chunk, H), jnp.float32),
            pltpu.SemaphoreType.DMA,
            pltpu.SemaphoreType.DMA,
        ],
    )
    def gather_k(t_hbm, src_hbm, dst_hbm, o1_hbm, o2_hbm,
                 idx1_v, rows1_v, idx2_v, rows2_v, sem1, sem2):
        wid = lax.axis_index("s") * 2 + lax.axis_index("c")
        base = wid * epw

        def body(i, _):
            off = base + i * chunk
            pltpu.sync_copy(src_hbm.at[pl.ds(off, chunk)], idx1_v)
            pltpu.sync_copy(dst_hbm.at[pl.ds(off, chunk)], idx2_v)
            c1 = pltpu.async_copy(t_hbm.at[idx1_v], rows1_v, sem1)
            c2 = pltpu.async_copy(t_hbm.at[idx2_v], rows2_v, sem2)
            c1.wait()
            c2.wait()
            pltpu.sync_copy(rows1_v, o1_hbm.at[pl.ds(off, chunk)])
            pltpu.sync_copy(rows2_v, o2_hbm.at[pl.ds(off, chunk)])
            return 0

        lax.fori_loop(0, nchunk, body, 0)

    return gather_k


def _make_scatter(E, N, H, n_workers, chunk):
    epw = E // n_workers
    nchunk = epw // chunk
    nsub = n_workers // 2
    npr = (N // nsub) & ~7
    tail = N - nsub * npr
    mesh = plsc.VectorSubcoreMesh(core_axis_name="c", subcore_axis_name="s")

    @functools.partial(
        pl.kernel,
        out_type=jax.ShapeDtypeStruct((2 * N, H), jnp.float32),
        mesh=mesh,
        scratch_types=[
            pltpu.VMEM((chunk,), jnp.int32),
            pltpu.VMEM((chunk, H), jnp.float32),
            pltpu.VMEM_SHARED((N, H), jnp.float32),
        ],
    )
    def scatter_k(gated_hbm, dst_hbm, zeros_hbm, out_hbm,
                  idx_v, rows_v, acc_sh):
        cid = lax.axis_index("c")
        sid = lax.axis_index("s")
        wid = sid * 2 + cid
        base = wid * epw
        pltpu.sync_copy(zeros_hbm.at[pl.ds(sid * npr, npr)],
                        acc_sh.at[pl.ds(sid * npr, npr)])
        if tail:
            @pl.when(sid == 0)
            def _():
                pltpu.sync_copy(zeros_hbm.at[pl.ds(nsub * npr, tail)],
                                acc_sh.at[pl.ds(nsub * npr, tail)])
        plsc.subcore_barrier()

        def body(i, _):
            off = base + i * chunk
            pltpu.sync_copy(dst_hbm.at[pl.ds(off, chunk)], idx_v)
            pltpu.sync_copy(gated_hbm.at[pl.ds(off, chunk)], rows_v)
            pltpu.sync_copy(rows_v, acc_sh.at[idx_v], add=True)
            return 0

        lax.fori_loop(0, nchunk, body, 0)
        plsc.subcore_barrier()
        pltpu.sync_copy(acc_sh.at[pl.ds(sid * npr, npr)],
                        out_hbm.at[pl.ds(cid * N + sid * npr, npr)])
        if tail:
            @pl.when(sid == 0)
            def _():
                pltpu.sync_copy(acc_sh.at[pl.ds(nsub * npr, tail)],
                                out_hbm.at[pl.ds(cid * N + nsub * npr, tail)])

    return scatter_k


def kernel(x, edge_d, params, edge_index):
    N, D = x.shape
    E = edge_index.shape[1]
    H = params['in'][0].shape[1]
    T = params['out2'][0].shape[1]
    depth = len(params['layers'])

    src = edge_index[0]
    dst = edge_index[1]

    NW = 32
    CH = 80
    BE = 1280
    BN = 1000
    assert E % (NW * CH) == 0 and E % BE == 0
    assert N % 16 == 0 and N % BN == 0

    gather_k = _make_gather(E, N, H, NW, CH)
    scatter_k = _make_scatter(E, N, H, NW, CH)
    zeros_nh = jnp.zeros((N, H), jnp.float32)

    f32 = jnp.float32
    row = lambda v: v.reshape(1, -1).astype(f32)

    ne_grid = (E // BE,)
    nn_grid = (N // BN,)
    eb = lambda: pl.BlockSpec((BE, H), lambda i: (i, 0))
    nb = lambda: pl.BlockSpec((BN, H), lambda i: (i, 0))

    feat = pl.pallas_call(
        _node_in_body,
        grid=nn_grid,
        in_specs=[nb(), _wspec((D, H)), _wspec((1, H))],
        out_specs=nb(),
        out_shape=jax.ShapeDtypeStruct((N, H), f32),
    )(x, params['in'][0], row(params['in'][1]))

    d = None
    out = None
    for l in range(depth):
        lp = params['layers'][l]
        gs, gd = gather_k(feat, src, dst)

        if l == 0:
            dnew, gated = pl.pallas_call(
                _edge0_body,
                grid=ne_grid,
                in_specs=[eb(), eb(),
                          pl.BlockSpec((BE, 1), lambda i: (i, 0)),
                          _wspec((1, H)), _wspec((1, H)),
                          _wspec((3 * H, H)), _wspec((1, H)),
                          _wspec((H, H)), _wspec((1, H)),
                          _wspec((H, 1)), _wspec((1, 1))],
                out_specs=[eb(), eb()],
                out_shape=[jax.ShapeDtypeStruct((E, H), f32)] * 2,
            )(gs, gd, edge_d, row(params['edge'][0][0]),
              row(params['edge'][1]),
              lp['m1'][0], row(lp['m1'][1]), lp['m2'][0], row(lp['m2'][1]),
              lp['soft'][0], lp['soft'][1].reshape(1, 1))
        else:
            dnew, gated = pl.pallas_call(
                _edge_body,
                grid=ne_grid,
                in_specs=[eb(), eb(), eb(),
                          _wspec((3 * H, H)), _wspec((1, H)),
                          _wspec((H, H)), _wspec((1, H)),
                          _wspec((H, 1)), _wspec((1, 1))],
                out_specs=[eb(), eb()],
                out_shape=[jax.ShapeDtypeStruct((E, H), f32)] * 2,
            )(gs, gd, d, lp['m1'][0], row(lp['m1'][1]),
              lp['m2'][0], row(lp['m2'][1]),
              lp['soft'][0], lp['soft'][1].reshape(1, 1))
        d = dnew

        msum2 = scatter_k(gated, dst, zeros_nh).reshape(2, N, H)

        if l + 1 < depth:
            feat = pl.pallas_call(
                _update_body,
                grid=nn_grid,
                in_specs=[pl.BlockSpec((2, BN, H), lambda i: (0, i, 0)),
                          nb(),
                          _wspec((H, H)), _wspec((1, H)),
                          _wspec((H, H)), _wspec((1, H))],
                out_specs=nb(),
                out_shape=jax.ShapeDtypeStruct((N, H), f32),
            )(msum2, feat, lp['u1'][0], row(lp['u1'][1]),
              lp['u2'][0], row(lp['u2'][1]))
        else:
            out = pl.pallas_call(
                _final_body,
                grid=nn_grid,
                in_specs=[pl.BlockSpec((2, BN, H), lambda i: (0, i, 0)),
                          nb(),
                          _wspec((H, H)), _wspec((1, H)),
                          _wspec((H, H)), _wspec((1, H)),
                          _wspec((H, H)), _wspec((1, H)),
                          _wspec((H, T)), _wspec((1, T))],
                out_specs=pl.BlockSpec((BN, T), lambda i: (i, 0)),
                out_shape=jax.ShapeDtypeStruct((N, T), f32),
            )(msum2, feat, lp['u1'][0], row(lp['u1'][1]),
              lp['u2'][0], row(lp['u2'][1]),
              params['out1'][0], row(params['out1'][1]),
              params['out2'][0], row(params['out2'][1]))
    return out
